# baseline (speedup 1.0000x reference)
import jax
import jax.numpy as jnp
from jax.experimental import pallas as pl

TQ = 256


def _warp_kernel(q_ref, k_ref, p_ref, o_ref):
    q = q_ref[0]
    k = k_ref[0]
    p = p_ref[0]

    qn = jnp.sum(q * q, axis=1, keepdims=True)
    kn = jnp.sum(k * k, axis=0, keepdims=True)

    mm = jnp.dot(q, k)
    d = -2.0 * mm
    d = d + qn
    d = d + kn

    m1 = jnp.min(d, axis=1, keepdims=True)
    e1 = d == m1
    d2 = jnp.where(e1, jnp.inf, d)
    m2 = jnp.min(d2, axis=1, keepdims=True)
    e2 = d2 == m2
    d3 = jnp.where(e2, jnp.inf, d2)
    t = jnp.min(d3, axis=1, keepdims=True)

    mm_hi = jnp.dot(q, k, precision="highest")
    dd = -2.0 * mm_hi
    dd = dd + qn
    dd = dd + kn
    dd = jnp.maximum(dd, 0.0)

    w = jnp.where(d <= t, jnp.minimum(jax.lax.rsqrt(dd), 1e10), 0.0)

    r = jnp.dot(w, p, precision="highest")
    o_ref[0] = q - r[:, 0:3] / r[:, 3:4]


def kernel(xyz1, xyz2, flow1):
    b, c, n1 = xyz1.shape
    n2 = xyz2.shape[2]
    keys = xyz1 + flow1
    queries = jnp.transpose(xyz2, (0, 2, 1))
    fpack = jnp.concatenate(
        [jnp.transpose(flow1, (0, 2, 1)),
         jnp.ones((b, n1, 1), jnp.float32)], axis=-1)

    out = pl.pallas_call(
        _warp_kernel,
        grid=(b, n2 // TQ),
        in_specs=[
            pl.BlockSpec((1, TQ, c), lambda i, j: (i, j, 0)),
            pl.BlockSpec((1, c, n1), lambda i, j: (i, 0, 0)),
            pl.BlockSpec((1, n1, 4), lambda i, j: (i, 0, 0)),
        ],
        out_specs=pl.BlockSpec((1, TQ, c), lambda i, j: (i, j, 0)),
        out_shape=jax.ShapeDtypeStruct((b, n2, c), jnp.float32),
    )(queries, keys, fpack)

    return jnp.transpose(out, (0, 2, 1))


# speedup vs baseline: 2.2554x; 2.2554x over previous
import jax
import jax.numpy as jnp
from jax.experimental import pallas as pl

TQ = 256


def _warp_kernel(q_ref, k_ref, p_ref, o_ref):
    q = q_ref[0]
    k = k_ref[0]
    p = p_ref[0]

    qn = jnp.sum(q * q, axis=1, keepdims=True)
    kn = jnp.sum(k * k, axis=0, keepdims=True)

    mm = jnp.dot(q, k)
    d = -2.0 * mm
    d = d + qn
    d = d + kn

    m1 = jnp.min(d, axis=1, keepdims=True)
    e1 = d == m1
    d2 = jnp.where(e1, jnp.inf, d)
    m2 = jnp.min(d2, axis=1, keepdims=True)
    e2 = d2 == m2
    d3 = jnp.where(e2, jnp.inf, d2)
    t = jnp.min(d3, axis=1, keepdims=True)

    dd = (k[0:1, :] - q[:, 0:1]) ** 2
    dd = dd + (k[1:2, :] - q[:, 1:2]) ** 2
    dd = dd + (k[2:3, :] - q[:, 2:3]) ** 2

    w = jnp.where(d <= t, jnp.minimum(jax.lax.rsqrt(dd), 1e10), 0.0)

    r = jnp.dot(w, p)
    o_ref[0] = q - r[:, 0:3] / r[:, 3:4]


def kernel(xyz1, xyz2, flow1):
    b, c, n1 = xyz1.shape
    n2 = xyz2.shape[2]
    keys = xyz1 + flow1
    queries = jnp.transpose(xyz2, (0, 2, 1))
    fpack = jnp.concatenate(
        [jnp.transpose(flow1, (0, 2, 1)),
         jnp.ones((b, n1, 1), jnp.float32)], axis=-1)

    out = pl.pallas_call(
        _warp_kernel,
        grid=(b, n2 // TQ),
        in_specs=[
            pl.BlockSpec((1, TQ, c), lambda i, j: (i, j, 0)),
            pl.BlockSpec((1, c, n1), lambda i, j: (i, 0, 0)),
            pl.BlockSpec((1, n1, 4), lambda i, j: (i, 0, 0)),
        ],
        out_specs=pl.BlockSpec((1, TQ, c), lambda i, j: (i, j, 0)),
        out_shape=jax.ShapeDtypeStruct((b, n2, c), jnp.float32),
    )(queries, keys, fpack)

    return jnp.transpose(out, (0, 2, 1))


# speedup vs baseline: 2.3783x; 1.0545x over previous
import jax
import jax.numpy as jnp
from jax.experimental import pallas as pl

TQ = 512


def _warp_kernel(q_ref, k_ref, p_ref, o_ref):
    q = q_ref[0]
    k = k_ref[0]
    p = p_ref[0]

    qn = jnp.sum(q * q, axis=1, keepdims=True)
    kn = jnp.sum(k * k, axis=0, keepdims=True)

    mm = jnp.dot(q, k)
    d = -2.0 * mm
    d = d + qn
    d = d + kn

    m1 = jnp.min(d, axis=1, keepdims=True)
    e1 = d == m1
    d2 = jnp.where(e1, jnp.inf, d)
    m2 = jnp.min(d2, axis=1, keepdims=True)
    e2 = d2 == m2
    d3 = jnp.where(e2, jnp.inf, d2)
    t = jnp.min(d3, axis=1, keepdims=True)

    dd = (k[0:1, :] - q[:, 0:1]) ** 2
    dd = dd + (k[1:2, :] - q[:, 1:2]) ** 2
    dd = dd + (k[2:3, :] - q[:, 2:3]) ** 2

    w = jnp.where(d <= t, jnp.minimum(jax.lax.rsqrt(dd), 1e10), 0.0)

    r = jnp.dot(w, p)
    o_ref[0] = q - r[:, 0:3] / r[:, 3:4]


def kernel(xyz1, xyz2, flow1):
    b, c, n1 = xyz1.shape
    n2 = xyz2.shape[2]
    keys = xyz1 + flow1
    queries = jnp.transpose(xyz2, (0, 2, 1))
    fpack = jnp.concatenate(
        [jnp.transpose(flow1, (0, 2, 1)),
         jnp.ones((b, n1, 1), jnp.float32)], axis=-1)

    out = pl.pallas_call(
        _warp_kernel,
        grid=(b, n2 // TQ),
        in_specs=[
            pl.BlockSpec((1, TQ, c), lambda i, j: (i, j, 0)),
            pl.BlockSpec((1, c, n1), lambda i, j: (i, 0, 0)),
            pl.BlockSpec((1, n1, 4), lambda i, j: (i, 0, 0)),
        ],
        out_specs=pl.BlockSpec((1, TQ, c), lambda i, j: (i, j, 0)),
        out_shape=jax.ShapeDtypeStruct((b, n2, c), jnp.float32),
    )(queries, keys, fpack)

    return jnp.transpose(out, (0, 2, 1))
